# baseline (speedup 1.0000x reference)
import jax
import jax.numpy as jnp
from jax import lax
from jax.experimental import pallas as pl
from jax.experimental.pallas import tpu as pltpu
from jax.experimental.pallas import tpu_sc as plsc

_B, _L, _D = 1024, 2048, 128
_N = _B * _L
_NC, _NS = 2, 16
_NW = _NC * _NS
_NPW = _N // _NW
_C = 256
_KSUB = _C // 128
_NBUF = 2
_IDXBLK = 8192
_NBLK = _NPW // _IDXBLK
_GRPS = _IDXBLK // (_NBUF * _C)


def _emb_body(idx_hbm, table_hbm, out_hbm,
              table_sh, idx_v, rows0, rows1,
              gsem0, gsem1, wsem0, wsem1):
    cid = lax.axis_index("c")
    sid = lax.axis_index("s")
    wid = sid * _NC + cid
    base = wid * _NPW
    rows = (rows0, rows1)
    gsem = (gsem0, gsem1)
    wsem = (wsem0, wsem1)

    @pl.when(sid == 0)
    def _fill_table():
        pltpu.sync_copy(table_hbm, table_sh)

    plsc.subcore_barrier()
    woff = sid * 5

    def blk(ob, carry):
        blk_off = base + ob * _IDXBLK
        pltpu.sync_copy(idx_hbm.at[pl.ds(blk_off, _IDXBLK)], idx_v)
        for i in range(_IDXBLK // 16):
            idx_v[pl.ds(i * 16, 16)] = idx_v[pl.ds(i * 16, 16)] + woff

        def grp(p, carry2):
            for b in range(_NBUF):
                loc = (p * _NBUF + b) * _C
                off = blk_off + loc

                @pl.when((ob > 0) | (p > 0))
                def _wait_prev_write():
                    pltpu.make_async_copy(
                        rows[b], out_hbm.at[pl.ds(off, _C)], wsem[b]).wait()

                for j in range(_KSUB):
                    pltpu.async_copy(
                        table_sh.at[idx_v.at[pl.ds(loc + j * 128, 128)]],
                        rows[b].at[pl.ds(j * 128, 128)],
                        gsem[b],
                    )
            for b in range(_NBUF):
                loc = (p * _NBUF + b) * _C
                off = blk_off + loc
                for j in range(_KSUB):
                    pltpu.make_async_copy(
                        table_sh.at[idx_v.at[pl.ds(loc + j * 128, 128)]],
                        rows[b].at[pl.ds(j * 128, 128)], gsem[b]).wait()
                pltpu.async_copy(rows[b], out_hbm.at[pl.ds(off, _C)], wsem[b])
            return carry2

        lax.fori_loop(0, _GRPS, grp, 0)
        return carry

    lax.fori_loop(0, _NBLK, blk, 0)
    for b in range(_NBUF):
        pltpu.make_async_copy(
            rows[b], out_hbm.at[pl.ds(base, _C)], wsem[b]).wait()


@jax.jit
def _emb(idx2, table):
    mesh = plsc.VectorSubcoreMesh(core_axis_name="c", subcore_axis_name="s")
    f = pl.kernel(
        _emb_body,
        mesh=mesh,
        out_type=jax.ShapeDtypeStruct((_N, _D), jnp.float32),
        scratch_types=[
            pltpu.VMEM_SHARED((5 * _NS, _D), jnp.float32),
            pltpu.VMEM((_IDXBLK,), jnp.int32),
            pltpu.VMEM((_C, _D), jnp.float32),
            pltpu.VMEM((_C, _D), jnp.float32),
            pltpu.SemaphoreType.DMA,
            pltpu.SemaphoreType.DMA,
            pltpu.SemaphoreType.DMA,
            pltpu.SemaphoreType.DMA,
        ],
    )
    return f(idx2, table)


def kernel(sequence_int, table):
    idx2 = sequence_int.reshape(_N)
    table_rep = jnp.tile(table, (_NS, 1))
    out = _emb(idx2, table_rep)
    return out.reshape(_B, _L, _D)


# speedup vs baseline: 1.5082x; 1.5082x over previous
import jax
import jax.numpy as jnp
from jax import lax
from jax.experimental import pallas as pl
from jax.experimental.pallas import tpu as pltpu
from jax.experimental.pallas import tpu_sc as plsc

_B, _L, _D = 1024, 2048, 128
_N = _B * _L
_NC, _NS = 2, 16
_NW = _NC * _NS
_NPW = _N // _NW
_C = 128
_NBUF = 4
_IDXBLK = 8192
_NBLK = _NPW // _IDXBLK
_GRPS = _IDXBLK // (_NBUF * _C)


def _emb_body(idx_hbm, table_hbm, out_hbm,
              table_sh, idxa, idxb, rows0, rows1, rows2, rows3,
              isema, isemb,
              gsem0, gsem1, gsem2, gsem3, wsem0, wsem1, wsem2, wsem3):
    cid = lax.axis_index("c")
    sid = lax.axis_index("s")
    wid = sid * _NC + cid
    base = wid * _NPW
    idxv = (idxa, idxb)
    isem = (isema, isemb)
    rows = (rows0, rows1, rows2, rows3)
    gsem = (gsem0, gsem1, gsem2, gsem3)
    wsem = (wsem0, wsem1, wsem2, wsem3)

    @pl.when(sid == 0)
    def _fill_table():
        pltpu.sync_copy(table_hbm, table_sh)

    plsc.subcore_barrier()
    woff = sid * 5

    pltpu.async_copy(idx_hbm.at[pl.ds(base, _IDXBLK)], idxv[0], isem[0])

    def sblk(sb, carry):
        for q in range(2):
            ob = sb * 2 + q
            blk_off = base + ob * _IDXBLK
            iv = idxv[q]
            pltpu.make_async_copy(
                idx_hbm.at[pl.ds(blk_off, _IDXBLK)], iv, isem[q]).wait()

            @pl.when(ob + 1 < _NBLK)
            def _prefetch_next():
                pltpu.async_copy(
                    idx_hbm.at[pl.ds(blk_off + _IDXBLK, _IDXBLK)],
                    idxv[1 - q], isem[1 - q])

            def grp(p, carry2):
                for b in range(_NBUF):
                    loc = (p * _NBUF + b) * _C
                    off = blk_off + loc
                    for i in range(_C // 16):
                        s = loc + i * 16
                        iv[pl.ds(s, 16)] = iv[pl.ds(s, 16)] + woff

                    @pl.when((ob > 0) | (p > 0))
                    def _wait_prev_write():
                        pltpu.make_async_copy(
                            rows[b], out_hbm.at[pl.ds(off, _C)],
                            wsem[b]).wait()

                    pltpu.async_copy(
                        table_sh.at[iv.at[pl.ds(loc, _C)]],
                        rows[b],
                        gsem[b],
                    )
                for b in range(_NBUF):
                    loc = (p * _NBUF + b) * _C
                    off = blk_off + loc
                    pltpu.make_async_copy(
                        table_sh.at[iv.at[pl.ds(loc, _C)]],
                        rows[b], gsem[b]).wait()
                    pltpu.async_copy(
                        rows[b], out_hbm.at[pl.ds(off, _C)], wsem[b])
                return carry2

            lax.fori_loop(0, _GRPS, grp, 0)
        return carry

    lax.fori_loop(0, _NBLK // 2, sblk, 0)
    for b in range(_NBUF):
        pltpu.make_async_copy(
            rows[b], out_hbm.at[pl.ds(base, _C)], wsem[b]).wait()


@jax.jit
def _emb(idx2, table):
    mesh = plsc.VectorSubcoreMesh(core_axis_name="c", subcore_axis_name="s")
    f = pl.kernel(
        _emb_body,
        mesh=mesh,
        out_type=jax.ShapeDtypeStruct((_N, _D), jnp.float32),
        scratch_types=[
            pltpu.VMEM_SHARED((5 * _NS, _D), jnp.float32),
            pltpu.VMEM((_IDXBLK,), jnp.int32),
            pltpu.VMEM((_IDXBLK,), jnp.int32),
            pltpu.VMEM((_C, _D), jnp.float32),
            pltpu.VMEM((_C, _D), jnp.float32),
            pltpu.VMEM((_C, _D), jnp.float32),
            pltpu.VMEM((_C, _D), jnp.float32),
            pltpu.SemaphoreType.DMA,
            pltpu.SemaphoreType.DMA,
            pltpu.SemaphoreType.DMA,
            pltpu.SemaphoreType.DMA,
            pltpu.SemaphoreType.DMA,
            pltpu.SemaphoreType.DMA,
            pltpu.SemaphoreType.DMA,
            pltpu.SemaphoreType.DMA,
            pltpu.SemaphoreType.DMA,
            pltpu.SemaphoreType.DMA,
        ],
    )
    return f(idx2, table)


def kernel(sequence_int, table):
    idx2 = sequence_int.reshape(_N)
    table_rep = jnp.tile(table, (_NS, 1))
    out = _emb(idx2, table_rep)
    return out.reshape(_B, _L, _D)


# speedup vs baseline: 1.5147x; 1.0043x over previous
import jax
import jax.numpy as jnp
from jax import lax
from jax.experimental import pallas as pl
from jax.experimental.pallas import tpu as pltpu
from jax.experimental.pallas import tpu_sc as plsc

_B, _L, _D = 1024, 2048, 128
_N = _B * _L
_NC, _NS = 2, 16
_NW = _NC * _NS
_NPW = _N // _NW
_C = 64
_NBUF = 8
_IDXBLK = 8192
_NBLK = _NPW // _IDXBLK
_GRPS = _IDXBLK // (_NBUF * _C)


def _emb_body(idx_hbm, table_hbm, out_hbm,
              table_sh, idxa, idxb,
              rows0, rows1, rows2, rows3, rows4, rows5, rows6, rows7,
              isema, isemb,
              gsem0, gsem1, gsem2, gsem3, gsem4, gsem5, gsem6, gsem7,
              wsem0, wsem1, wsem2, wsem3, wsem4, wsem5, wsem6, wsem7):
    cid = lax.axis_index("c")
    sid = lax.axis_index("s")
    wid = sid * _NC + cid
    base = wid * _NPW
    idxv = (idxa, idxb)
    isem = (isema, isemb)
    rows = (rows0, rows1, rows2, rows3, rows4, rows5, rows6, rows7)
    gsem = (gsem0, gsem1, gsem2, gsem3, gsem4, gsem5, gsem6, gsem7)
    wsem = (wsem0, wsem1, wsem2, wsem3, wsem4, wsem5, wsem6, wsem7)

    @pl.when(sid == 0)
    def _fill_table():
        pltpu.sync_copy(table_hbm, table_sh)

    plsc.subcore_barrier()
    woff = sid * 5

    pltpu.async_copy(idx_hbm.at[pl.ds(base, _IDXBLK)], idxv[0], isem[0])

    def sblk(sb, carry):
        for q in range(2):
            ob = sb * 2 + q
            blk_off = base + ob * _IDXBLK
            iv = idxv[q]
            pltpu.make_async_copy(
                idx_hbm.at[pl.ds(blk_off, _IDXBLK)], iv, isem[q]).wait()

            @pl.when(ob + 1 < _NBLK)
            def _prefetch_next():
                pltpu.async_copy(
                    idx_hbm.at[pl.ds(blk_off + _IDXBLK, _IDXBLK)],
                    idxv[1 - q], isem[1 - q])

            def grp(p, carry2):
                for b in range(_NBUF):
                    loc = (p * _NBUF + b) * _C
                    off = blk_off + loc
                    for i in range(_C // 16):
                        s = loc + i * 16
                        iv[pl.ds(s, 16)] = iv[pl.ds(s, 16)] + woff

                    @pl.when((ob > 0) | (p > 0))
                    def _wait_prev_write():
                        pltpu.make_async_copy(
                            rows[b], out_hbm.at[pl.ds(off, _C)],
                            wsem[b]).wait()

                    pltpu.async_copy(
                        table_sh.at[iv.at[pl.ds(loc, _C)]],
                        rows[b],
                        gsem[b],
                    )
                for b in range(_NBUF):
                    loc = (p * _NBUF + b) * _C
                    off = blk_off + loc
                    pltpu.make_async_copy(
                        table_sh.at[iv.at[pl.ds(loc, _C)]],
                        rows[b], gsem[b]).wait()
                    pltpu.async_copy(
                        rows[b], out_hbm.at[pl.ds(off, _C)], wsem[b])
                return carry2

            lax.fori_loop(0, _GRPS, grp, 0)
        return carry

    lax.fori_loop(0, _NBLK // 2, sblk, 0)
    for b in range(_NBUF):
        pltpu.make_async_copy(
            rows[b], out_hbm.at[pl.ds(base, _C)], wsem[b]).wait()


@jax.jit
def _emb(idx2, table):
    mesh = plsc.VectorSubcoreMesh(core_axis_name="c", subcore_axis_name="s")
    f = pl.kernel(
        _emb_body,
        mesh=mesh,
        out_type=jax.ShapeDtypeStruct((_N, _D), jnp.float32),
        scratch_types=[
            pltpu.VMEM_SHARED((5 * _NS, _D), jnp.float32),
            pltpu.VMEM((_IDXBLK,), jnp.int32),
            pltpu.VMEM((_IDXBLK,), jnp.int32),
            pltpu.VMEM((_C, _D), jnp.float32),
            pltpu.VMEM((_C, _D), jnp.float32),
            pltpu.VMEM((_C, _D), jnp.float32),
            pltpu.VMEM((_C, _D), jnp.float32),
            pltpu.VMEM((_C, _D), jnp.float32),
            pltpu.VMEM((_C, _D), jnp.float32),
            pltpu.VMEM((_C, _D), jnp.float32),
            pltpu.VMEM((_C, _D), jnp.float32),
            pltpu.SemaphoreType.DMA,
            pltpu.SemaphoreType.DMA,
            pltpu.SemaphoreType.DMA,
            pltpu.SemaphoreType.DMA,
            pltpu.SemaphoreType.DMA,
            pltpu.SemaphoreType.DMA,
            pltpu.SemaphoreType.DMA,
            pltpu.SemaphoreType.DMA,
            pltpu.SemaphoreType.DMA,
            pltpu.SemaphoreType.DMA,
            pltpu.SemaphoreType.DMA,
            pltpu.SemaphoreType.DMA,
            pltpu.SemaphoreType.DMA,
            pltpu.SemaphoreType.DMA,
            pltpu.SemaphoreType.DMA,
            pltpu.SemaphoreType.DMA,
            pltpu.SemaphoreType.DMA,
            pltpu.SemaphoreType.DMA,
        ],
    )
    return f(idx2, table)


def kernel(sequence_int, table):
    idx2 = sequence_int.reshape(_N)
    table_rep = jnp.tile(table, (_NS, 1))
    out = _emb(idx2, table_rep)
    return out.reshape(_B, _L, _D)
